# Initial kernel scaffold; baseline (speedup 1.0000x reference)
import jax
import jax.numpy as jnp
from jax.experimental import pallas as pl


def kernel(tgt_tokens, tgt_pos, edge_index, value_table, coord_table, pos_table, Wqkv, Wo, W1, W2, Wgen):
    raise NotImplementedError("write your pallas kernel here")


# speedup vs baseline: 25.4562x; 25.4562x over previous
import functools

import jax
import jax.numpy as jnp
from jax import lax
from jax.experimental import pallas as pl
from jax.experimental.pallas import tpu as pltpu
from jax.experimental.pallas import tpu_sc as plsc

N = 10000
E = 320000
H = 8
DK = 16
D = H * DK
DFF = 512
VOCAB = 1000
MAXPOS = 4096
NL = 2

NB = 10
BLK = N // NB
EB = 80
EBE = 80
NW = 32
EPW = E // NW
NEB = EPW // EBE
NTB = N // EB
NZB = N // EBE
SCALE = 1.0 / (DK ** 0.5)

_mesh = plsc.VectorSubcoreMesh(core_axis_name="c", subcore_axis_name="s")


def _embed_body(tok_hbm, tpos_hbm, vtab, ctab, ptab, x_hbm,
                tki, pix, cix, vb, pb, cb, xb, sem):
    c = lax.axis_index("c")
    s = lax.axis_index("s")
    wid = c * 16 + s
    three = jnp.full((16,), 3, jnp.int32)
    for jj in range(4):
        b = wid + 32 * jj

        @pl.when(b < NTB)
        def _():
            off = pl.multiple_of(b * EB, 8)
            pltpu.sync_copy(tok_hbm.at[pl.ds(off, EB)], tki)
            pltpu.sync_copy(tpos_hbm.at[pl.ds(off, EB)], pix)

            def ixb(i, carry):
                sl = pl.ds(i * 16, 16)
                pv = pix[sl]
                cix[sl] = lax.rem(pv, three)
                pix[sl] = lax.div(pv, three)
                return carry

            lax.fori_loop(0, EB // 16, ixb, 0)
            cv = pltpu.async_copy(vtab.at[tki], vb, sem)
            cp = pltpu.async_copy(ptab.at[pix], pb, sem)
            cc = pltpu.async_copy(ctab.at[cix], cb, sem)
            cv.wait()
            cp.wait()
            cc.wait()

            def rowb(i, carry):
                for h in range(H):
                    sl = pl.ds(h * 16, 16)
                    xb[i, sl] = vb[i, sl] + pb[i, sl] + cb[i, sl]
                return carry

            lax.fori_loop(0, EB, rowb, 0)
            pltpu.sync_copy(xb, x_hbm.at[pl.ds(off, EB)])


_embed_call = pl.kernel(
    _embed_body,
    out_type=jax.ShapeDtypeStruct((N, D), jnp.float32),
    mesh=_mesh,
    compiler_params=pltpu.CompilerParams(needs_layout_passes=False),
    scratch_types=[
        pltpu.VMEM((EB,), jnp.int32),
        pltpu.VMEM((EB,), jnp.int32),
        pltpu.VMEM((EB,), jnp.int32),
        pltpu.VMEM((EB, D), jnp.float32),
        pltpu.VMEM((EB, D), jnp.float32),
        pltpu.VMEM((EB, D), jnp.float32),
        pltpu.VMEM((EB, D), jnp.float32),
        pltpu.SemaphoreType.DMA,
    ],
)


NHALF = 5000
NH = 5040
NHB = NH // EBE


def _edge_body(q_hbm, k_hbm, v_hbm, src_hbm, dst_hbm, zout_hbm, wout_hbm,
               sidx, didx, ldx, kb, qb, shz, shw, sem):
    c = lax.axis_index("c")
    s = lax.axis_index("s")

    def zbody(i, carry):
        for h in range(H):
            qb[i, pl.ds(h * 16, 16)] = jnp.zeros((16,), jnp.float32)
        return carry

    lax.fori_loop(0, EBE, zbody, 0)
    for jj in range(4):
        b = jnp.minimum(s + 16 * jj, NHB - 1)
        off = pl.multiple_of(b * EBE, 8)
        pltpu.async_copy(qb, shz.at[pl.ds(off, EBE)], sem).wait()
        pltpu.async_copy(qb, shw.at[pl.ds(off, EBE)], sem).wait()

    plsc.subcore_barrier()

    base = s * (2 * EPW)
    lo = c * NHALF

    def batch(j, carry):
        off = pl.multiple_of(base + j * EBE, 8)
        ci = pltpu.async_copy(src_hbm.at[pl.ds(off, EBE)], sidx, sem)
        cj = pltpu.async_copy(dst_hbm.at[pl.ds(off, EBE)], didx, sem)
        ci.wait()
        cj.wait()

        def locb(i, carry2):
            sl = pl.ds(i * 16, 16)
            dv = didx[sl] - lo
            ok = (dv >= 0) & (dv < NHALF)
            ldx[sl] = jnp.where(ok, dv, jnp.full((16,), NHALF, jnp.int32))
            return carry2

        lax.fori_loop(0, EBE // 16, locb, 0)
        ck = pltpu.async_copy(k_hbm.at[sidx], kb, sem)
        cq = pltpu.async_copy(q_hbm.at[didx], qb, sem)
        ck.wait()
        cq.wait()

        def edge(e, icarry):
            for h in range(H):
                sl = pl.ds(h * 16, 16)
                sc = jnp.sum(kb[e, sl] * qb[e, sl])
                kb[e, sl] = jnp.exp(jnp.full((16,), sc))
            return icarry

        lax.fori_loop(0, EBE, edge, 0)

        cv = pltpu.async_copy(v_hbm.at[sidx], qb, sem)
        cv.wait()

        def scale(e, icarry):
            for h in range(H):
                sl = pl.ds(h * 16, 16)
                qb[e, sl] = qb[e, sl] * kb[e, sl]
            return icarry

        lax.fori_loop(0, EBE, scale, 0)
        cs = pltpu.async_copy(qb, shz.at[ldx], sem, add=True)
        cs.wait()
        cw = pltpu.async_copy(kb, shw.at[ldx], sem, add=True)
        cw.wait()
        return carry

    lax.fori_loop(0, 2 * EPW // EBE, batch, 0)
    plsc.subcore_barrier()

    for jj in range(4):
        b = jnp.minimum(s + 16 * jj, NHB - 1)
        off = pl.multiple_of(b * EBE, 8)
        gout = pl.multiple_of(c * NH + b * EBE, 8)
        pltpu.async_copy(shz.at[pl.ds(off, EBE)],
                         zout_hbm.at[pl.ds(gout, EBE)], sem).wait()
        pltpu.async_copy(shw.at[pl.ds(off, EBE)],
                         wout_hbm.at[pl.ds(gout, EBE)], sem).wait()


_edge_call = pl.kernel(
    _edge_body,
    out_type=[jax.ShapeDtypeStruct((2 * NH, D), jnp.float32),
              jax.ShapeDtypeStruct((2 * NH, D), jnp.float32)],
    mesh=_mesh,
    compiler_params=pltpu.CompilerParams(needs_layout_passes=False),
    scratch_types=[
        pltpu.VMEM((EBE,), jnp.int32),
        pltpu.VMEM((EBE,), jnp.int32),
        pltpu.VMEM((EBE,), jnp.int32),
        pltpu.VMEM((EBE, D), jnp.float32),
        pltpu.VMEM((EBE, D), jnp.float32),
        pltpu.VMEM_SHARED((NH, D), jnp.float32),
        pltpu.VMEM_SHARED((NH, D), jnp.float32),
        pltpu.SemaphoreType.DMA,
    ],
)


def _ln(x):
    mu = jnp.mean(x, axis=-1, keepdims=True)
    var = jnp.mean((x - mu) ** 2, axis=-1, keepdims=True)
    return (x - mu) * lax.rsqrt(var + 1e-5)


def _pre_body(x_ref, w_ref, q_ref, k_ref, v_ref):
    xn = _ln(x_ref[...])
    qkv = jnp.dot(xn, w_ref[...], preferred_element_type=jnp.float32)
    q_ref[...] = qkv[:, :D] * SCALE
    k_ref[...] = qkv[:, D:2 * D]
    v_ref[...] = qkv[:, 2 * D:]


_pre_call = pl.pallas_call(
    _pre_body,
    grid=(NB,),
    in_specs=[
        pl.BlockSpec((BLK, D), lambda i: (i, 0)),
        pl.BlockSpec((D, 3 * D), lambda i: (0, 0)),
    ],
    out_specs=[
        pl.BlockSpec((BLK, D), lambda i: (i, 0)),
        pl.BlockSpec((BLK, D), lambda i: (i, 0)),
        pl.BlockSpec((BLK, D), lambda i: (i, 0)),
    ],
    out_shape=[jax.ShapeDtypeStruct((N, D), jnp.float32)] * 3,
)


def _post_body(x_ref, zp_ref, wp_ref, wo_ref, w1_ref, w2_ref, o_ref):
    zn = zp_ref[...] / (wp_ref[...] + 1e-9)
    x = x_ref[...] + jnp.dot(zn, wo_ref[...],
                             preferred_element_type=jnp.float32)
    xn = _ln(x)
    hdn = jnp.maximum(
        jnp.dot(xn, w1_ref[...], preferred_element_type=jnp.float32), 0.0)
    o_ref[...] = x + jnp.dot(hdn, w2_ref[...],
                             preferred_element_type=jnp.float32)


_post_call = pl.pallas_call(
    _post_body,
    grid=(NB,),
    in_specs=[
        pl.BlockSpec((BLK, D), lambda i: (i, 0)),
        pl.BlockSpec((BLK, D), lambda i: (i, 0)),
        pl.BlockSpec((BLK, D), lambda i: (i, 0)),
        pl.BlockSpec((D, D), lambda i: (0, 0)),
        pl.BlockSpec((D, DFF), lambda i: (0, 0)),
        pl.BlockSpec((DFF, D), lambda i: (0, 0)),
    ],
    out_specs=pl.BlockSpec((BLK, D), lambda i: (i, 0)),
    out_shape=jax.ShapeDtypeStruct((N, D), jnp.float32),
)


def _gen_body(x_ref, wg_ref, o_ref):
    xn = _ln(x_ref[...])
    lg = jnp.dot(xn, wg_ref[...], preferred_element_type=jnp.float32)
    m = jnp.max(lg, axis=-1, keepdims=True)
    ex = jnp.exp(lg - m)
    o_ref[...] = lg - m - jnp.log(jnp.sum(ex, axis=-1, keepdims=True))


_gen_call = pl.pallas_call(
    _gen_body,
    grid=(NB,),
    in_specs=[
        pl.BlockSpec((BLK, D), lambda i: (i, 0)),
        pl.BlockSpec((D, VOCAB), lambda i: (0, 0)),
    ],
    out_specs=pl.BlockSpec((BLK, VOCAB), lambda i: (i, 0)),
    out_shape=jax.ShapeDtypeStruct((N, VOCAB), jnp.float32),
)


def kernel(tgt_tokens, tgt_pos, edge_index, value_table, coord_table,
           pos_table, Wqkv, Wo, W1, W2, Wgen):
    tok = tgt_tokens.astype(jnp.int32)
    tpos = tgt_pos.astype(jnp.int32)
    src = edge_index[0].astype(jnp.int32)
    dst = edge_index[1].astype(jnp.int32)
    x = _embed_call(tok, tpos, value_table, coord_table, pos_table)
    for i in range(NL):
        q, k, v = _pre_call(x, Wqkv[i])
        zp, wp = _edge_call(q, k, v, src, dst)
        zp = zp.reshape(2, NH, D)
        wp = wp.reshape(2, NH, D)
        z = jnp.concatenate([zp[0, :NHALF], zp[1, :NHALF]], axis=0)
        w = jnp.concatenate([wp[0, :NHALF], wp[1, :NHALF]], axis=0)
        x = _post_call(x, z, w, Wo[i], W1[i], W2[i])
    return _gen_call(x, Wgen)
